# Initial kernel scaffold; baseline (speedup 1.0000x reference)
import functools

import jax
import jax.numpy as jnp
from jax import lax
from jax.experimental import pallas as pl
from jax.experimental.pallas import tpu as pltpu
from jax.experimental.pallas import tpu_sc as plsc

F32 = jnp.float32


def _silu(x):
    return x * (1.0 / (1.0 + jnp.exp(-x)))


def _node_pre(h, wa, wb, bn=2000):
    n, d = h.shape

    def body(h_r, wa_r, wb_r, a_r, b_r):
        hblk = h_r[...]
        a_r[...] = jnp.dot(hblk, wa_r[...], preferred_element_type=F32)
        b_r[...] = jnp.dot(hblk, wb_r[...], preferred_element_type=F32)

    return pl.pallas_call(
        body,
        grid=(n // bn,),
        in_specs=[
            pl.BlockSpec((bn, d), lambda i: (i, 0)),
            pl.BlockSpec((d, d), lambda i: (0, 0)),
            pl.BlockSpec((d, d), lambda i: (0, 0)),
        ],
        out_specs=[
            pl.BlockSpec((bn, d), lambda i: (i, 0)),
            pl.BlockSpec((bn, d), lambda i: (i, 0)),
        ],
        out_shape=[
            jax.ShapeDtypeStruct((n, d), F32),
            jax.ShapeDtypeStruct((n, d), F32),
        ],
    )(h, wa, wb)


def _gather_stage(a, b, coordp, row, col):
    n, d = a.shape
    e = row.shape[0]
    info = plsc.get_sparse_core_info()
    nc, ns = info.num_cores, info.num_subcores
    nw = nc * ns
    ew = e // nw
    c = 80
    nch = ew // c
    mesh = plsc.VectorSubcoreMesh(core_axis_name="c", subcore_axis_name="s")

    @functools.partial(
        pl.kernel,
        mesh=mesh,
        out_type=[
            jax.ShapeDtypeStruct((e, d), F32),
            jax.ShapeDtypeStruct((e, 16), F32),
        ],
        scratch_types=[
            pltpu.VMEM((c,), jnp.int32),
            pltpu.VMEM((c,), jnp.int32),
            pltpu.VMEM((c, d), F32),
            pltpu.VMEM((c, d), F32),
            pltpu.VMEM((c, 16), F32),
            pltpu.VMEM((c, 16), F32),
            pltpu.SemaphoreType.DMA,
            pltpu.SemaphoreType.DMA,
            pltpu.SemaphoreType.DMA,
            pltpu.SemaphoreType.DMA,
        ],
    )
    def k(a_h, b_h, cp_h, row_h, col_h, g0_h, cd_h,
          idx_r, idx_c, abuf, bbuf, crbuf, ccbuf, sa, sb, sr, sc):
        wid = lax.axis_index("s") * nc + lax.axis_index("c")
        wbase = wid * ew

        def chunk(ch, carry):
            base = wbase + ch * c
            pltpu.sync_copy(row_h.at[pl.ds(base, c)], idx_r)
            pltpu.sync_copy(col_h.at[pl.ds(base, c)], idx_c)
            ca = pltpu.async_copy(a_h.at[idx_r], abuf, sa)
            cb = pltpu.async_copy(b_h.at[idx_c], bbuf, sb)
            cr = pltpu.async_copy(cp_h.at[idx_r], crbuf, sr)
            cc = pltpu.async_copy(cp_h.at[idx_c], ccbuf, sc)
            ca.wait()
            cb.wait()
            cr.wait()
            cc.wait()

            def rbody(i, cy):
                crbuf[i, :] = crbuf[i, :] - ccbuf[i, :]
                for j in range(d // 16):
                    sl = pl.ds(j * 16, 16)
                    abuf[i, sl] = abuf[i, sl] + bbuf[i, sl]
                return cy

            lax.fori_loop(0, c, rbody, 0)
            pltpu.sync_copy(abuf, g0_h.at[pl.ds(base, c)])
            pltpu.sync_copy(crbuf, cd_h.at[pl.ds(base, c)])
            return carry

        lax.fori_loop(0, nch, chunk, 0)

    return k(a, b, coordp, row, col)


def _edge_stage(g0, cdp, ea, wr, we, eb1, ew2, eb2, cw1, cb1, cw2t, be=2000):
    e, d = g0.shape
    de = ea.shape[1]

    def body(g_r, cd_r, ea_r, wr_r, we_r, eb1_r, ew2_r, eb2_r, cw1_r, cb1_r,
             cw2_r, ef_r, t_r):
        cd = cd_r[...]
        radial = jnp.sum(cd * cd, axis=1, keepdims=True)
        pre1 = (g_r[...] + radial * wr_r[...]
                + jnp.dot(ea_r[...], we_r[...], preferred_element_type=F32)
                + eb1_r[...])
        m = _silu(pre1)
        ef = _silu(jnp.dot(m, ew2_r[...], preferred_element_type=F32)
                   + eb2_r[...])
        cm = _silu(jnp.dot(ef, cw1_r[...], preferred_element_type=F32)
                   + cb1_r[...])
        s = jnp.sum(cm * cw2_r[...], axis=1, keepdims=True)
        lane = lax.broadcasted_iota(jnp.int32, (be, 16), 1)
        t_r[...] = cd * s + jnp.where(lane == 3, 1.0, 0.0).astype(F32)
        ef_r[...] = ef

    zero2 = lambda i: (0, 0)
    return pl.pallas_call(
        body,
        grid=(e // be,),
        in_specs=[
            pl.BlockSpec((be, d), lambda i: (i, 0)),
            pl.BlockSpec((be, 16), lambda i: (i, 0)),
            pl.BlockSpec((be, de), lambda i: (i, 0)),
            pl.BlockSpec((1, d), zero2),
            pl.BlockSpec((de, d), zero2),
            pl.BlockSpec((1, d), zero2),
            pl.BlockSpec((d, d), zero2),
            pl.BlockSpec((1, d), zero2),
            pl.BlockSpec((d, d), zero2),
            pl.BlockSpec((1, d), zero2),
            pl.BlockSpec((1, d), zero2),
        ],
        out_specs=[
            pl.BlockSpec((be, d), lambda i: (i, 0)),
            pl.BlockSpec((be, 16), lambda i: (i, 0)),
        ],
        out_shape=[
            jax.ShapeDtypeStruct((e, d), F32),
            jax.ShapeDtypeStruct((e, 16), F32),
        ],
    )(g0, cdp, ea, wr, we, eb1, ew2, eb2, cw1, cb1, cw2t)


def _scatter_stage(ef, t16, row, n):
    e, d = ef.shape
    info = plsc.get_sparse_core_info()
    nc, ns = info.num_cores, info.num_subcores
    nw = nc * ns
    ewk = e // nw
    c = 80
    nch = ewk // c
    rpt = n // ns
    zc = 125
    nzc = rpt // zc
    mesh = plsc.VectorSubcoreMesh(core_axis_name="c", subcore_axis_name="s")

    @functools.partial(
        pl.kernel,
        mesh=mesh,
        out_type=[
            jax.ShapeDtypeStruct((nc, n, d), F32),
            jax.ShapeDtypeStruct((nc, n, 16), F32),
        ],
        scratch_types=[
            pltpu.VMEM((c,), jnp.int32),
            pltpu.VMEM((c, d), F32),
            pltpu.VMEM((c, 16), F32),
            pltpu.VMEM((zc, d), F32),
            pltpu.VMEM((rpt, 16), F32),
            pltpu.VMEM_SHARED((n, d), F32),
            pltpu.VMEM_SHARED((n, 16), F32),
        ],
    )
    def k(ef_h, t_h, row_h, hp_h, tp_h,
          idx_v, efbuf, tbuf, zbuf, zbuf16, hacc, tacc):
        cid = lax.axis_index("c")
        sid = lax.axis_index("s")
        wid = sid * nc + cid
        rbase = sid * rpt

        def zb(i, cy):
            for j in range(d // 16):
                zbuf[i, pl.ds(j * 16, 16)] = jnp.zeros((16,), F32)
            return cy

        lax.fori_loop(0, zc, zb, 0)

        def zb2(i, cy):
            zbuf16[i, :] = jnp.zeros((16,), F32)
            return cy

        lax.fori_loop(0, rpt, zb2, 0)

        def zs(kk, cy):
            pltpu.sync_copy(zbuf, hacc.at[pl.ds(rbase + kk * zc, zc)])
            return cy

        lax.fori_loop(0, nzc, zs, 0)
        pltpu.sync_copy(zbuf16, tacc.at[pl.ds(rbase, rpt)])
        plsc.subcore_barrier()

        wbase = wid * ewk

        def chunk(ch, carry):
            base = wbase + ch * c
            pltpu.sync_copy(row_h.at[pl.ds(base, c)], idx_v)
            pltpu.sync_copy(ef_h.at[pl.ds(base, c)], efbuf)
            pltpu.sync_copy(t_h.at[pl.ds(base, c)], tbuf)
            pltpu.sync_copy(efbuf, hacc.at[idx_v], add=True)
            pltpu.sync_copy(tbuf, tacc.at[idx_v], add=True)
            return carry

        lax.fori_loop(0, nch, chunk, 0)
        plsc.subcore_barrier()
        pltpu.sync_copy(hacc.at[pl.ds(rbase, rpt)],
                        hp_h.at[cid, pl.ds(rbase, rpt)])
        pltpu.sync_copy(tacc.at[pl.ds(rbase, rpt)],
                        tp_h.at[cid, pl.ds(rbase, rpt)])

    return k(ef, t16, row)


def _node_stage(h, hp0, hp1, tp0, tp1, coordp, nw1a, nw1b, nb1, nw2, nb2,
                bn=2000):
    n, d = h.shape

    def body(h_r, hp0_r, hp1_r, tp0_r, tp1_r, cp_r, w1a_r, w1b_r, b1_r,
             w2_r, b2_r, ho_r, co_r):
        hblk = h_r[...]
        agg = hp0_r[...] + hp1_r[...]
        t = tp0_r[...] + tp1_r[...]
        pre = (jnp.dot(hblk, w1a_r[...], preferred_element_type=F32)
               + jnp.dot(agg, w1b_r[...], preferred_element_type=F32)
               + b1_r[...])
        hid = _silu(pre)
        ho_r[...] = hblk + jnp.dot(hid, w2_r[...],
                                   preferred_element_type=F32) + b2_r[...]
        lane = lax.broadcasted_iota(jnp.int32, (bn, 16), 1)
        cnt = jnp.sum(jnp.where(lane == 3, t, 0.0), axis=1, keepdims=True)
        co_r[...] = cp_r[...] + t / jnp.maximum(cnt, 1.0)

    zero2 = lambda i: (0, 0)
    return pl.pallas_call(
        body,
        grid=(n // bn,),
        in_specs=[
            pl.BlockSpec((bn, d), lambda i: (i, 0)),
            pl.BlockSpec((bn, d), lambda i: (i, 0)),
            pl.BlockSpec((bn, d), lambda i: (i, 0)),
            pl.BlockSpec((bn, 16), lambda i: (i, 0)),
            pl.BlockSpec((bn, 16), lambda i: (i, 0)),
            pl.BlockSpec((bn, 16), lambda i: (i, 0)),
            pl.BlockSpec((d, d), zero2),
            pl.BlockSpec((d, d), zero2),
            pl.BlockSpec((1, d), zero2),
            pl.BlockSpec((d, d), zero2),
            pl.BlockSpec((1, d), zero2),
        ],
        out_specs=[
            pl.BlockSpec((bn, d), lambda i: (i, 0)),
            pl.BlockSpec((bn, 16), lambda i: (i, 0)),
        ],
        out_shape=[
            jax.ShapeDtypeStruct((n, d), F32),
            jax.ShapeDtypeStruct((n, 16), F32),
        ],
    )(h, hp0, hp1, tp0, tp1, coordp, nw1a, nw1b, nb1, nw2, nb2)


def kernel(h, edge_index, coord, edge_attr, ew1, eb1, ew2, eb2, nw1, nb1,
           nw2, nb2, cw1, cb1, cw2):
    n, d = h.shape
    row = edge_index[0]
    col = edge_index[1]
    coordp = jnp.pad(coord, ((0, 0), (0, 16 - coord.shape[1])))
    wa = ew1[:d]
    wb = ew1[d:2 * d]
    wr = ew1[2 * d:2 * d + 1]
    we = ew1[2 * d + 1:]
    a, b = _node_pre(h, wa, wb)
    g0, cdp = _gather_stage(a, b, coordp, row, col)
    ef, t16 = _edge_stage(g0, cdp, edge_attr, wr, we, eb1.reshape(1, -1),
                          ew2, eb2.reshape(1, -1), cw1, cb1.reshape(1, -1),
                          cw2.reshape(1, -1))
    hpart, tpart = _scatter_stage(ef, t16, row, n)
    hout, cpout = _node_stage(h, hpart[0], hpart[1], tpart[0], tpart[1],
                              coordp, nw1[:d], nw1[d:], nb1.reshape(1, -1),
                              nw2, nb2.reshape(1, -1))
    return hout, cpout[:, :3], edge_attr


# speedup vs baseline: 2.7382x; 2.7382x over previous
import functools

import jax
import jax.numpy as jnp
from jax import lax
from jax.experimental import pallas as pl
from jax.experimental.pallas import tpu as pltpu
from jax.experimental.pallas import tpu_sc as plsc

F32 = jnp.float32


def _silu(x):
    return x * (1.0 / (1.0 + jnp.exp(-x)))


def _node_pre(h, wa, wb, bn=2000):
    n, d = h.shape

    def body(h_r, wa_r, wb_r, a_r, b_r):
        hblk = h_r[...]
        a_r[...] = jnp.dot(hblk, wa_r[...], preferred_element_type=F32)
        b_r[...] = jnp.dot(hblk, wb_r[...], preferred_element_type=F32)

    return pl.pallas_call(
        body,
        grid=(n // bn,),
        in_specs=[
            pl.BlockSpec((bn, d), lambda i: (i, 0)),
            pl.BlockSpec((d, d), lambda i: (0, 0)),
            pl.BlockSpec((d, d), lambda i: (0, 0)),
        ],
        out_specs=[
            pl.BlockSpec((bn, d), lambda i: (i, 0)),
            pl.BlockSpec((bn, d), lambda i: (i, 0)),
        ],
        out_shape=[
            jax.ShapeDtypeStruct((n, d), F32),
            jax.ShapeDtypeStruct((n, d), F32),
        ],
    )(h, wa, wb)


def _gather_stage(a, b, cx, cy, cz, row, col):
    n, d = a.shape
    e = row.shape[0]
    info = plsc.get_sparse_core_info()
    nc, ns = info.num_cores, info.num_subcores
    nw = nc * ns
    ew = e // nw
    c = 80
    nch = ew // c
    mesh = plsc.VectorSubcoreMesh(core_axis_name="c", subcore_axis_name="s")

    @functools.partial(
        pl.kernel,
        mesh=mesh,
        compiler_params=pltpu.CompilerParams(needs_layout_passes=False),
        out_type=[
            jax.ShapeDtypeStruct((e, d), F32),
            jax.ShapeDtypeStruct((e, 16), F32),
        ],
        scratch_types=[
            pltpu.VMEM((c,), jnp.int32),
            pltpu.VMEM((c,), jnp.int32),
            pltpu.VMEM((n,), F32),
            pltpu.VMEM((n,), F32),
            pltpu.VMEM((n,), F32),
            pltpu.VMEM((c, d), F32),
            pltpu.VMEM((c, d), F32),
            pltpu.VMEM((c, 16), F32),
            pltpu.SemaphoreType.DMA,
            pltpu.SemaphoreType.DMA,
        ],
    )
    def k(a_h, b_h, cx_h, cy_h, cz_h, row_h, col_h, g0_h, cd_h,
          idx_r, idx_c, cxv, cyv, czv, abuf, bbuf, cdbuf, sa, sb):
        wid = lax.axis_index("s") * nc + lax.axis_index("c")
        wbase = wid * ew
        pltpu.sync_copy(cx_h, cxv)
        pltpu.sync_copy(cy_h, cyv)
        pltpu.sync_copy(cz_h, czv)

        def zb(i, cy_):
            cdbuf[i, :] = jnp.zeros((16,), F32)
            return cy_

        lax.fori_loop(0, c, zb, 0)
        lane = lax.broadcasted_iota(jnp.int32, (16,), 0)
        czero = jnp.zeros((16,), jnp.int32)

        def chunk(ch, carry):
            base = wbase + ch * c
            pltpu.sync_copy(row_h.at[pl.ds(base, c)], idx_r)
            pltpu.sync_copy(col_h.at[pl.ds(base, c)], idx_c)
            ca = pltpu.async_copy(a_h.at[idx_r], abuf, sa)
            cb = pltpu.async_copy(b_h.at[idx_c], bbuf, sb)

            for g in range(c // 16):
                sl = pl.ds(g * 16, 16)
                ir = idx_r[sl]
                ic = idx_c[sl]
                rows = g * 16 + lane
                dx = plsc.load_gather(cxv, [ir]) - plsc.load_gather(cxv, [ic])
                plsc.store_scatter(cdbuf, [rows, czero], dx)
                dy = plsc.load_gather(cyv, [ir]) - plsc.load_gather(cyv, [ic])
                plsc.store_scatter(cdbuf, [rows, czero + 1], dy)
                dz = plsc.load_gather(czv, [ir]) - plsc.load_gather(czv, [ic])
                plsc.store_scatter(cdbuf, [rows, czero + 2], dz)

            ca.wait()
            cb.wait()

            def rbody(i, cy_):
                for j in range(d // 16):
                    sl2 = pl.ds(j * 16, 16)
                    abuf[i, sl2] = abuf[i, sl2] + bbuf[i, sl2]
                return cy_

            lax.fori_loop(0, c, rbody, 0)
            pltpu.sync_copy(abuf, g0_h.at[pl.ds(base, c)])
            pltpu.sync_copy(cdbuf, cd_h.at[pl.ds(base, c)])
            return carry

        lax.fori_loop(0, nch, chunk, 0)

    return k(a, b, cx, cy, cz, row, col)


def _edge_stage(g0, cdp, ea, wr, we, eb1, ew2, eb2, cw1, cb1, cw2t, be=2000):
    e, d = g0.shape
    de = ea.shape[1]

    def body(g_r, cd_r, ea_r, wr_r, we_r, eb1_r, ew2_r, eb2_r, cw1_r, cb1_r,
             cw2_r, ef_r, t_r):
        cd = cd_r[...]
        radial = jnp.sum(cd * cd, axis=1, keepdims=True)
        pre1 = (g_r[...] + radial * wr_r[...]
                + jnp.dot(ea_r[...], we_r[...], preferred_element_type=F32)
                + eb1_r[...])
        m = _silu(pre1)
        ef = _silu(jnp.dot(m, ew2_r[...], preferred_element_type=F32)
                   + eb2_r[...])
        cm = _silu(jnp.dot(ef, cw1_r[...], preferred_element_type=F32)
                   + cb1_r[...])
        s = jnp.sum(cm * cw2_r[...], axis=1, keepdims=True)
        lane = lax.broadcasted_iota(jnp.int32, (be, 16), 1)
        t_r[...] = cd * s + jnp.where(lane == 3, 1.0, 0.0).astype(F32)
        ef_r[...] = ef

    zero2 = lambda i: (0, 0)
    return pl.pallas_call(
        body,
        grid=(e // be,),
        in_specs=[
            pl.BlockSpec((be, d), lambda i: (i, 0)),
            pl.BlockSpec((be, 16), lambda i: (i, 0)),
            pl.BlockSpec((be, de), lambda i: (i, 0)),
            pl.BlockSpec((1, d), zero2),
            pl.BlockSpec((de, d), zero2),
            pl.BlockSpec((1, d), zero2),
            pl.BlockSpec((d, d), zero2),
            pl.BlockSpec((1, d), zero2),
            pl.BlockSpec((d, d), zero2),
            pl.BlockSpec((1, d), zero2),
            pl.BlockSpec((1, d), zero2),
        ],
        out_specs=[
            pl.BlockSpec((be, d), lambda i: (i, 0)),
            pl.BlockSpec((be, 16), lambda i: (i, 0)),
        ],
        out_shape=[
            jax.ShapeDtypeStruct((e, d), F32),
            jax.ShapeDtypeStruct((e, 16), F32),
        ],
    )(g0, cdp, ea, wr, we, eb1, ew2, eb2, cw1, cb1, cw2t)


def _scatter_stage(ef, t16, row, n):
    e, d = ef.shape
    info = plsc.get_sparse_core_info()
    nc, ns = info.num_cores, info.num_subcores
    nw = nc * ns
    ewk = e // nw
    c = 80
    nch = ewk // c
    segr = 5056
    sega = 5120
    nseg = 2
    nzc = sega // 8 // ns
    mesh = plsc.VectorSubcoreMesh(core_axis_name="c", subcore_axis_name="s")

    @functools.partial(
        pl.kernel,
        mesh=mesh,
        compiler_params=pltpu.CompilerParams(needs_layout_passes=False),
        out_type=[
            jax.ShapeDtypeStruct((nc, nseg * sega, d), F32),
            jax.ShapeDtypeStruct((nc, nseg * sega, d), F32),
        ],
        scratch_types=[
            pltpu.VMEM((c,), jnp.int32),
            pltpu.VMEM((c,), jnp.int32),
            pltpu.VMEM((c, d), F32),
            pltpu.VMEM((c, 16), F32),
            pltpu.VMEM((c, d), F32),
            pltpu.VMEM((8, d), F32),
            pltpu.VMEM_SHARED((sega, d), F32),
        ],
    )
    def k(ef_h, t_h, row_h, hp_h, tp_h,
          idx_v, idx2_v, efbuf, tbuf16, tbuf, zbuf, hacc):
        cid = lax.axis_index("c")
        sid = lax.axis_index("s")
        wid = sid * nc + cid
        wbase = wid * ewk

        for i in range(8):
            for j in range(d // 16):
                zbuf[i, pl.ds(j * 16, 16)] = jnp.zeros((16,), F32)

        def zt(i, cy):
            for j in range(d // 16):
                tbuf[i, pl.ds(j * 16, 16)] = jnp.zeros((16,), F32)
            return cy

        lax.fori_loop(0, c, zt, 0)

        def index_map(segbase):
            for g in range(c // 16):
                sl = pl.ds(g * 16, 16)
                iv = idx_v[sl] - segbase
                ok = (iv >= 0) & (iv < segr)
                idx2_v[sl] = jnp.where(ok, iv, segr)

        def zero_acc():
            def zs(i, cy):
                pltpu.sync_copy(zbuf, hacc.at[pl.ds((sid + i * ns) * 8, 8)])
                return cy
            lax.fori_loop(0, nzc, zs, 0)

        def writeback(dst, soff):
            def wb(i, cy):
                off = (sid + i * ns) * 8
                pltpu.sync_copy(hacc.at[pl.ds(off, 8)],
                                dst.at[cid, pl.ds(soff + off, 8)])
                return cy
            lax.fori_loop(0, nzc, wb, 0)

        for sg in range(nseg):
            zero_acc()
            plsc.subcore_barrier()
            pltpu.sync_copy(row_h.at[pl.ds(wbase, c)], idx_v)
            index_map(sg * segr)

            def chunk_a(ch, carry):
                base = wbase + ch * c
                pltpu.sync_copy(ef_h.at[pl.ds(base, c)], efbuf)
                pltpu.sync_copy(efbuf, hacc.at[idx2_v], add=True)
                pltpu.sync_copy(row_h.at[pl.ds(base + c, c)], idx_v)
                index_map(sg * segr)
                return carry

            lax.fori_loop(0, nch - 1, chunk_a, 0)
            base = wbase + (nch - 1) * c
            pltpu.sync_copy(ef_h.at[pl.ds(base, c)], efbuf)
            pltpu.sync_copy(efbuf, hacc.at[idx2_v], add=True)
            plsc.subcore_barrier()
            writeback(hp_h, sg * sega)
            plsc.subcore_barrier()

        for sg in range(nseg):
            zero_acc()
            plsc.subcore_barrier()
            pltpu.sync_copy(row_h.at[pl.ds(wbase, c)], idx_v)
            index_map(sg * segr)

            def chunk_b(ch, carry):
                base = wbase + ch * c
                pltpu.sync_copy(t_h.at[pl.ds(base, c)], tbuf16)

                def cp(i, cy):
                    tbuf[i, pl.ds(0, 16)] = tbuf16[i, :]
                    return cy

                lax.fori_loop(0, c, cp, 0)
                pltpu.sync_copy(tbuf, hacc.at[idx2_v], add=True)
                pltpu.sync_copy(row_h.at[pl.ds(base + c, c)], idx_v)
                index_map(sg * segr)
                return carry

            lax.fori_loop(0, nch - 1, chunk_b, 0)
            base = wbase + (nch - 1) * c
            pltpu.sync_copy(t_h.at[pl.ds(base, c)], tbuf16)

            def cp2(i, cy):
                tbuf[i, pl.ds(0, 16)] = tbuf16[i, :]
                return cy

            lax.fori_loop(0, c, cp2, 0)
            pltpu.sync_copy(tbuf, hacc.at[idx2_v], add=True)
            plsc.subcore_barrier()
            writeback(tp_h, sg * sega)
            plsc.subcore_barrier()

    hp, tp = k(ef, t16, row)
    hpart = jnp.concatenate([hp[:, :segr], hp[:, sega:sega + n - segr]],
                            axis=1)
    tpart = jnp.concatenate([tp[:, :segr, :16], tp[:, sega:sega + n - segr, :16]],
                            axis=1)
    return hpart, tpart


def _node_stage(h, hp0, hp1, tp0, tp1, coordp, nw1a, nw1b, nb1, nw2, nb2,
                bn=2000):
    n, d = h.shape

    def body(h_r, hp0_r, hp1_r, tp0_r, tp1_r, cp_r, w1a_r, w1b_r, b1_r,
             w2_r, b2_r, ho_r, co_r):
        hblk = h_r[...]
        agg = hp0_r[...] + hp1_r[...]
        t = tp0_r[...] + tp1_r[...]
        pre = (jnp.dot(hblk, w1a_r[...], preferred_element_type=F32)
               + jnp.dot(agg, w1b_r[...], preferred_element_type=F32)
               + b1_r[...])
        hid = _silu(pre)
        ho_r[...] = hblk + jnp.dot(hid, w2_r[...],
                                   preferred_element_type=F32) + b2_r[...]
        lane = lax.broadcasted_iota(jnp.int32, (bn, 16), 1)
        cnt = jnp.sum(jnp.where(lane == 3, t, 0.0), axis=1, keepdims=True)
        co_r[...] = cp_r[...] + t / jnp.maximum(cnt, 1.0)

    zero2 = lambda i: (0, 0)
    return pl.pallas_call(
        body,
        grid=(n // bn,),
        in_specs=[
            pl.BlockSpec((bn, d), lambda i: (i, 0)),
            pl.BlockSpec((bn, d), lambda i: (i, 0)),
            pl.BlockSpec((bn, d), lambda i: (i, 0)),
            pl.BlockSpec((bn, 16), lambda i: (i, 0)),
            pl.BlockSpec((bn, 16), lambda i: (i, 0)),
            pl.BlockSpec((bn, 16), lambda i: (i, 0)),
            pl.BlockSpec((d, d), zero2),
            pl.BlockSpec((d, d), zero2),
            pl.BlockSpec((1, d), zero2),
            pl.BlockSpec((d, d), zero2),
            pl.BlockSpec((1, d), zero2),
        ],
        out_specs=[
            pl.BlockSpec((bn, d), lambda i: (i, 0)),
            pl.BlockSpec((bn, 16), lambda i: (i, 0)),
        ],
        out_shape=[
            jax.ShapeDtypeStruct((n, d), F32),
            jax.ShapeDtypeStruct((n, 16), F32),
        ],
    )(h, hp0, hp1, tp0, tp1, coordp, nw1a, nw1b, nb1, nw2, nb2)


def kernel(h, edge_index, coord, edge_attr, ew1, eb1, ew2, eb2, nw1, nb1,
           nw2, nb2, cw1, cb1, cw2):
    n, d = h.shape
    row = edge_index[0]
    col = edge_index[1]
    coordp = jnp.pad(coord, ((0, 0), (0, 16 - coord.shape[1])))
    wa = ew1[:d]
    wb = ew1[d:2 * d]
    wr = ew1[2 * d:2 * d + 1]
    we = ew1[2 * d + 1:]
    a, b = _node_pre(h, wa, wb)
    g0, cdp = _gather_stage(a, b, coord[:, 0], coord[:, 1], coord[:, 2],
                            row, col)
    ef, t16 = _edge_stage(g0, cdp, edge_attr, wr, we, eb1.reshape(1, -1),
                          ew2, eb2.reshape(1, -1), cw1, cb1.reshape(1, -1),
                          cw2.reshape(1, -1))
    hpart, tpart = _scatter_stage(ef, t16, row, n)
    hout, cpout = _node_stage(h, hpart[0], hpart[1], tpart[0], tpart[1],
                              coordp, nw1[:d], nw1[d:], nb1.reshape(1, -1),
                              nw2, nb2.reshape(1, -1))
    return hout, cpout[:, :3], edge_attr
